# baseline (speedup 1.0000x reference)
import functools

import jax
import jax.numpy as jnp
from jax import lax
from jax.experimental import pallas as pl
from jax.experimental.pallas import tpu as pltpu
from jax.experimental.pallas import tpu_sc as plsc

V = 1000000
D = 20
B = 16384
BF = 2 * B
NFETCH = 2 * BF

_info = plsc.get_sparse_core_info()
_NC, _NS = _info.num_cores, _info.num_subcores
_NW = _NC * _NS
_B_PER_W = NFETCH // _NW
_HALF_W = _B_PER_W // 2


def _make_gather():
    mesh = plsc.VectorSubcoreMesh(core_axis_name="c", subcore_axis_name="s")

    @functools.partial(
        pl.kernel,
        mesh=mesh,
        compiler_params=pltpu.CompilerParams(use_tc_tiling_on_sc=False),
        out_type=jax.ShapeDtypeStruct((NFETCH, D), jnp.float32),
        scratch_types=[
            pltpu.VMEM((_HALF_W,), jnp.int32),
            pltpu.VMEM((_HALF_W,), jnp.int32),
            pltpu.VMEM((_HALF_W, D), jnp.float32),
            pltpu.VMEM((_HALF_W, D), jnp.float32),
            pltpu.SemaphoreType.DMA,
        ],
    )
    def gather_k(table_hbm, idx_hbm, out_hbm, idx_a, idx_b, rows_a, rows_b,
                 sem):
        wid = lax.axis_index("s") * _NC + lax.axis_index("c")
        base = wid * _B_PER_W
        pltpu.sync_copy(idx_hbm.at[pl.ds(base, _HALF_W)], idx_a)
        pltpu.sync_copy(idx_hbm.at[pl.ds(base + _HALF_W, _HALF_W)], idx_b)
        ca = pltpu.async_copy(table_hbm.at[idx_a], rows_a, sem)
        cb = pltpu.async_copy(table_hbm.at[idx_b], rows_b, sem)
        ca.wait()
        cb.wait()
        pltpu.sync_copy(rows_a, out_hbm.at[pl.ds(base, _HALF_W)])
        pltpu.sync_copy(rows_b, out_hbm.at[pl.ds(base + _HALF_W, _HALF_W)])

    return gather_k


_gather = _make_gather()


_BLK = 2048


def _mlp_body(strips_ref, d4_ref, wa_ref, wb_ref, b1_ref, w2_ref, b2_ref,
              out_ref):
    strip0 = strips_ref[:, :40]
    strip1 = strips_ref[:, 40:]
    d4 = d4_ref[...]
    d0 = d4[:, 0:1]
    d1 = d4[:, 1:2]
    z1 = jnp.broadcast_to(b1_ref[...], (_BLK, D))
    for s in range(5):
        pa = jnp.dot(strip0, wa_ref[s * 40:(s + 1) * 40, :],
                     preferred_element_type=jnp.float32)
        pb = jnp.dot(strip1, wb_ref[s * 40:(s + 1) * 40, :],
                     preferred_element_type=jnp.float32)
        z1 = z1 + jnp.where(d0 == s, pa, 0.0) + jnp.where(d1 == s, pb, 0.0)
    a1 = jax.nn.sigmoid(z1)
    z2 = jnp.sum(a1 * w2_ref[...], axis=1, keepdims=True) + b2_ref[...]
    out_ref[...] = jax.nn.sigmoid(z2)


def kernel(x, embedding, W1, b1, W2, b2):
    v = x.astype(jnp.int32).reshape(-1)
    i0 = (6 * v) // 5
    idx2 = jnp.stack([i0, i0 + 1], axis=1).reshape(-1)
    d4 = (v % 5).reshape(B, 2)

    windows = _gather(embedding, idx2)
    strips = windows.reshape(B, 4 * D)

    w1t = W1.T
    shifted_a = []
    shifted_b = []
    for s in range(5):
        sa = jnp.zeros((40, D), jnp.float32)
        sa = sa.at[4 * s:4 * s + D, :].set(w1t[:D])
        sb = jnp.zeros((40, D), jnp.float32)
        sb = sb.at[4 * s:4 * s + D, :].set(w1t[D:])
        shifted_a.append(sa)
        shifted_b.append(sb)
    wa = jnp.concatenate(shifted_a, axis=0)
    wb = jnp.concatenate(shifted_b, axis=0)

    grid = B // _BLK
    out = pl.pallas_call(
        _mlp_body,
        grid=(grid,),
        in_specs=[
            pl.BlockSpec((_BLK, 4 * D), lambda i: (i, 0)),
            pl.BlockSpec((_BLK, 2), lambda i: (i, 0)),
            pl.BlockSpec((200, D), lambda i: (0, 0)),
            pl.BlockSpec((200, D), lambda i: (0, 0)),
            pl.BlockSpec((1, D), lambda i: (0, 0)),
            pl.BlockSpec((1, D), lambda i: (0, 0)),
            pl.BlockSpec((1, 1), lambda i: (0, 0)),
        ],
        out_specs=pl.BlockSpec((_BLK, 1), lambda i: (i, 0)),
        out_shape=jax.ShapeDtypeStruct((B, 1), jnp.float32),
    )(strips, d4, wa, wb, b1.reshape(1, D), W2.reshape(1, D),
      b2.reshape(1, 1))
    return out


# speedup vs baseline: 1.3557x; 1.3557x over previous
import functools

import jax
import jax.numpy as jnp
from jax import lax
from jax.experimental import pallas as pl
from jax.experimental.pallas import tpu as pltpu
from jax.experimental.pallas import tpu_sc as plsc

V = 1000000
D = 20
B = 16384
BF = 2 * B
PD = 2 * D

_info = plsc.get_sparse_core_info()
_NC, _NS = _info.num_cores, _info.num_subcores
_NW = _NC * _NS
_B_PER_W = BF // _NW


def _make_gather():
    mesh = plsc.VectorSubcoreMesh(core_axis_name="c", subcore_axis_name="s")

    @functools.partial(
        pl.kernel,
        mesh=mesh,
        compiler_params=pltpu.CompilerParams(use_tc_tiling_on_sc=False),
        out_type=jax.ShapeDtypeStruct((BF, PD), jnp.float32),
        scratch_types=[
            pltpu.VMEM((_B_PER_W,), jnp.int32),
            pltpu.VMEM((_B_PER_W, PD), jnp.float32),
            pltpu.SemaphoreType.DMA,
        ],
    )
    def gather_k(table_hbm, idx_hbm, out_hbm, idx_v, rows_v, sem):
        wid = lax.axis_index("s") * _NC + lax.axis_index("c")
        base = wid * _B_PER_W
        pltpu.sync_copy(idx_hbm.at[pl.ds(base, _B_PER_W)], idx_v)
        pltpu.async_copy(table_hbm.at[idx_v], rows_v, sem).wait()
        pltpu.sync_copy(rows_v, out_hbm.at[pl.ds(base, _B_PER_W)])

    return gather_k


_gather = _make_gather()

_BLK = 2048


def _mlp_body(strips_ref, par_ref, wa_ref, wb_ref, b1_ref, w2_ref, b2_ref,
              out_ref):
    strip0 = strips_ref[:, :PD]
    strip1 = strips_ref[:, PD:]
    par = par_ref[...]
    p0 = par[:, 0:1]
    p1 = par[:, 1:2]
    z1 = jnp.broadcast_to(b1_ref[...], (_BLK, D))
    for s in range(2):
        pa = jnp.dot(strip0, wa_ref[s * PD:(s + 1) * PD, :],
                     preferred_element_type=jnp.float32)
        pb = jnp.dot(strip1, wb_ref[s * PD:(s + 1) * PD, :],
                     preferred_element_type=jnp.float32)
        z1 = z1 + jnp.where(p0 == s, pa, 0.0) + jnp.where(p1 == s, pb, 0.0)
    a1 = jax.nn.sigmoid(z1)
    z2 = jnp.sum(a1 * w2_ref[...], axis=1, keepdims=True) + b2_ref[...]
    out_ref[...] = jax.nn.sigmoid(z2)


def kernel(x, embedding, W1, b1, W2, b2):
    table = embedding.reshape(V // 2, PD)
    v = x.astype(jnp.int32).reshape(-1)
    pidx = v // 2
    par = (v % 2).reshape(B, 2)

    rows = _gather(table, pidx)
    strips = windows = rows.reshape(B, 2 * PD)

    w1t = W1.T
    shifted_a = []
    shifted_b = []
    for s in range(2):
        sa = jnp.zeros((PD, D), jnp.float32)
        sa = sa.at[D * s:D * s + D, :].set(w1t[:D])
        sb = jnp.zeros((PD, D), jnp.float32)
        sb = sb.at[D * s:D * s + D, :].set(w1t[D:])
        shifted_a.append(sa)
        shifted_b.append(sb)
    wa = jnp.concatenate(shifted_a, axis=0)
    wb = jnp.concatenate(shifted_b, axis=0)

    grid = B // _BLK
    out = pl.pallas_call(
        _mlp_body,
        grid=(grid,),
        in_specs=[
            pl.BlockSpec((_BLK, 2 * PD), lambda i: (i, 0)),
            pl.BlockSpec((_BLK, 2), lambda i: (i, 0)),
            pl.BlockSpec((2 * PD, D), lambda i: (0, 0)),
            pl.BlockSpec((2 * PD, D), lambda i: (0, 0)),
            pl.BlockSpec((1, D), lambda i: (0, 0)),
            pl.BlockSpec((1, D), lambda i: (0, 0)),
            pl.BlockSpec((1, 1), lambda i: (0, 0)),
        ],
        out_specs=pl.BlockSpec((_BLK, 1), lambda i: (i, 0)),
        out_shape=jax.ShapeDtypeStruct((B, 1), jnp.float32),
    )(strips, par, wa, wb, b1.reshape(1, D), W2.reshape(1, D),
      b2.reshape(1, 1))
    return out
